# Initial kernel scaffold; baseline (speedup 1.0000x reference)
import jax
import jax.numpy as jnp
from jax.experimental import pallas as pl


def kernel(x, edge_index, W1, b1, W2, b2, W3, b3):
    raise NotImplementedError("write your pallas kernel here")


# speedup vs baseline: 11.7128x; 11.7128x over previous
import functools

import jax
import jax.numpy as jnp
from jax import lax
from jax.experimental import pallas as pl
from jax.experimental.pallas import tpu as pltpu
from jax.experimental.pallas import tpu_sc as plsc

_N = 10000
_E = 320000
_D = 128
_NC = 2
_NS = 16
_NW = _NC * _NS
_EPT = _E // _NW
_C = 80
_CHUNKS = _EPT // _C
_RPT = 624
_RLAST = _N - 15 * _RPT


def _sc_mesh():
    return plsc.VectorSubcoreMesh(core_axis_name="c", subcore_axis_name="s")


@functools.partial(
    pl.kernel,
    out_type=jax.ShapeDtypeStruct((_NC, _N), jnp.float32),
    mesh=_sc_mesh(),
    scratch_types=[
        pltpu.VMEM((_C,), jnp.int32),
        pltpu.VMEM((_C,), jnp.float32),
        pltpu.VMEM_SHARED((_N,), jnp.float32),
    ],
)
def _sc_deg(dst_hbm, zeros_hbm, out_hbm, dstv, ones, acc):
    c = lax.axis_index("c")
    s = lax.axis_index("s")
    for k in range(_C // 16):
        ones[pl.ds(k * 16, 16)] = jnp.ones((16,), jnp.float32)

    @pl.when(s == 0)
    def _():
        pltpu.sync_copy(zeros_hbm, acc)

    plsc.subcore_barrier()

    wid = c * _NS + s

    def body(i, carry):
        base = wid * _EPT + i * _C
        pltpu.sync_copy(dst_hbm.at[pl.ds(base, _C)], dstv)
        pltpu.sync_copy(ones, acc.at[dstv], add=True)
        return carry

    lax.fori_loop(0, _CHUNKS, body, 0)
    plsc.subcore_barrier()

    @pl.when(s == 0)
    def _():
        pltpu.sync_copy(acc, out_hbm.at[c])


@functools.partial(
    pl.kernel,
    out_type=jax.ShapeDtypeStruct((_NC, _N, _D), jnp.float32),
    mesh=_sc_mesh(),
    scratch_types=[
        pltpu.VMEM((_C,), jnp.int32),
        pltpu.VMEM((_C,), jnp.int32),
        pltpu.VMEM((_C, _D), jnp.float32),
        pltpu.VMEM_SHARED((_N, _D), jnp.float32),
        pltpu.SemaphoreType.DMA,
    ],
)
def _sc_agg(y_hbm, src_hbm, dst_hbm, zeros_hbm, out_hbm,
            srcv, dstv, rows, acc, sem):
    c = lax.axis_index("c")
    s = lax.axis_index("s")

    @pl.when(s < _NS - 1)
    def _():
        r0 = s * _RPT
        pltpu.sync_copy(zeros_hbm.at[pl.ds(r0, _RPT)],
                        acc.at[pl.ds(r0, _RPT)])

    @pl.when(s == _NS - 1)
    def _():
        r0 = 15 * _RPT
        pltpu.sync_copy(zeros_hbm.at[pl.ds(r0, _RLAST)],
                        acc.at[pl.ds(r0, _RLAST)])

    plsc.subcore_barrier()

    wid = c * _NS + s

    def body(i, carry):
        base = wid * _EPT + i * _C
        pltpu.sync_copy(src_hbm.at[pl.ds(base, _C)], srcv)
        pltpu.sync_copy(dst_hbm.at[pl.ds(base, _C)], dstv)
        pltpu.async_copy(y_hbm.at[srcv], rows, sem).wait()
        pltpu.sync_copy(rows, acc.at[dstv], add=True)
        return carry

    lax.fori_loop(0, _CHUNKS, body, 0)
    plsc.subcore_barrier()

    @pl.when(s < _NS - 1)
    def _():
        r0 = s * _RPT
        pltpu.sync_copy(acc.at[pl.ds(r0, _RPT)],
                        out_hbm.at[c].at[pl.ds(r0, _RPT)])

    @pl.when(s == _NS - 1)
    def _():
        r0 = 15 * _RPT
        pltpu.sync_copy(acc.at[pl.ds(r0, _RLAST)],
                        out_hbm.at[c].at[pl.ds(r0, _RLAST)])


def _tc1_body(x_ref, w_ref, degp_ref, y_ref, dis_ref):
    d = degp_ref[0] + degp_ref[1] + 1.0
    dis = lax.rsqrt(d)
    dis_ref[...] = dis
    y_ref[...] = dis * jnp.dot(x_ref[...], w_ref[...],
                               preferred_element_type=jnp.float32)


def _tc1(x, w, degp):
    return pl.pallas_call(
        _tc1_body,
        out_shape=(
            jax.ShapeDtypeStruct((_N, _D), jnp.float32),
            jax.ShapeDtypeStruct((_N, 1), jnp.float32),
        ),
    )(x, w, degp)


def _tc2_body(agg_ref, y_ref, dis_ref, b_ref, w_ref, out_ref):
    dis = dis_ref[...]
    z = dis * (agg_ref[0] + agg_ref[1] + y_ref[...]) + b_ref[...]
    h = jnp.maximum(z, 0.0)
    out_ref[...] = dis * jnp.dot(h, w_ref[...],
                                 preferred_element_type=jnp.float32)


def _tc2(agg, y, dis, b, w):
    return pl.pallas_call(
        _tc2_body,
        out_shape=jax.ShapeDtypeStruct((_N, _D), jnp.float32),
    )(agg, y, dis, b, w)


def _tc3_body(agg_ref, y_ref, dis_ref, b_ref, out_ref):
    out_ref[...] = (dis_ref[...] * (agg_ref[0] + agg_ref[1] + y_ref[...])
                    + b_ref[...])


def _tc3(agg, y, dis, b):
    return pl.pallas_call(
        _tc3_body,
        out_shape=jax.ShapeDtypeStruct((_N, _D), jnp.float32),
    )(agg, y, dis, b)


def kernel(x, edge_index, W1, b1, W2, b2, W3, b3):
    src = edge_index[0]
    dst = edge_index[1]
    zeros_n = jnp.zeros((_N,), jnp.float32)
    zeros_nd = jnp.zeros((_N, _D), jnp.float32)

    degp = _sc_deg(dst, zeros_n)
    y1, dis = _tc1(x, W1, degp.reshape(_NC, _N, 1))
    a1 = _sc_agg(y1, src, dst, zeros_nd)
    y2 = _tc2(a1, y1, dis, b1.reshape(1, _D), W2)
    a2 = _sc_agg(y2, src, dst, zeros_nd)
    y3 = _tc2(a2, y2, dis, b2.reshape(1, _D), W3)
    a3 = _sc_agg(y3, src, dst, zeros_nd)
    return _tc3(a3, y3, dis, b3.reshape(1, _D))
